# baseline (speedup 1.0000x reference)
import functools

import jax
import jax.numpy as jnp
from jax import lax
from jax.experimental import pallas as pl
from jax.experimental.pallas import tpu as pltpu
from jax.experimental.pallas import tpu_sc as plsc

_TN = 256
_PW = 2736
_CH = 304


def _rn_bf16(x):
    return x.astype(jnp.bfloat16).astype(jnp.float32)


def _rowsum_sq(a):
    sq = a * a
    t = sq[:, :128] + sq[:, 128:]
    acc = t[:, 0:8]
    for v in range(1, 16):
        acc = acc + t[:, 8 * v:8 * (v + 1)]
    b = acc[:, 0:4] + acc[:, 4:8]
    c = b[:, 0:2] + b[:, 2:4]
    return c[:, 0:1] + c[:, 1:2]


def _zsq_body(z_ref, o_ref):
    o_ref[...] = jnp.reshape(jnp.transpose(_rowsum_sq(z_ref[...])), (-1,))


def _esq_body(cb_ref, o_ref):
    o_ref[...] = _rowsum_sq(cb_ref[...])


def _row_norms(z, cbp):
    n, d = z.shape
    kp = cbp.shape[0]
    zsq = pl.pallas_call(
        _zsq_body,
        grid=(n // _TN,),
        in_specs=[pl.BlockSpec((_TN, d), lambda i: (i, 0))],
        out_specs=pl.BlockSpec((_TN,), lambda i: (i,)),
        out_shape=jax.ShapeDtypeStruct((n,), jnp.float32),
    )(z)
    esq = pl.pallas_call(
        _esq_body,
        grid=(kp // _CH,),
        in_specs=[pl.BlockSpec((_CH, d), lambda i: (i, 0))],
        out_specs=pl.BlockSpec((_CH, 1), lambda i: (i, 0)),
        out_shape=jax.ShapeDtypeStruct((kp, 1), jnp.float32),
    )(cbp)
    return zsq, esq


def _vq_body(scale, z_ref, cb_ref, zsq_ref, esq_ref, idx_ref, loss_ref,
             bval_ref, bidx_ref, bdist_ref, lsum_ref):
    j = pl.program_id(0)
    i = pl.program_id(1)
    nj = pl.num_programs(0)
    ni = pl.num_programs(1)

    z = z_ref[...]
    zsq = zsq_ref[...][None, :]

    m_run = None
    gi_run = None
    io = lax.broadcasted_iota(jnp.int32, (_CH, _TN), 0)
    for c in range(_PW // _CH):
        cbc = cb_ref[pl.ds(c * _CH, _CH), :]
        esq_c = esq_ref[pl.ds(c * _CH, _CH), :]
        mm = lax.dot_general(cbc, z, (((1,), (1,)), ((), ())),
                             preferred_element_type=jnp.float32)
        dist = (zsq + esq_c) - 2.0 * mm
        m_c = jnp.min(dist, axis=0)
        gi_c = jnp.min(jnp.where(dist == m_c[None, :], io, jnp.int32(_CH)),
                       axis=0) + (j * _PW + c * _CH)
        if m_run is None:
            m_run, gi_run = m_c, gi_c
        else:
            upd = m_c < m_run
            gi_run = jnp.where(upd, gi_c, gi_run)
            m_run = jnp.where(upd, m_c, m_run)

    @pl.when(j == 0)
    def _first():
        bval_ref[pl.ds(i, 1), :] = _rn_bf16(m_run)[None, :]
        bidx_ref[pl.ds(i, 1), :] = gi_run[None, :]
        bdist_ref[pl.ds(i, 1), :] = m_run[None, :]

    @pl.when(j != 0)
    def _merge():
        av = bval_ref[pl.ds(i, 1), :]
        bi = bidx_ref[pl.ds(i, 1), :]
        m2 = m_run[None, :]
        gi2 = gi_run[None, :]
        better = m2 < av
        take = better | ((m2 == av) & (gi2 < bi))
        bidx_ref[pl.ds(i, 1), :] = jnp.where(take, gi2, bi)
        bdist_ref[pl.ds(i, 1), :] = jnp.where(take, m2,
                                              bdist_ref[pl.ds(i, 1), :])
        bval_ref[pl.ds(i, 1), :] = jnp.where(better, _rn_bf16(m2), av)

    @pl.when(j == nj - 1)
    def _fin():
        idx_ref[...] = jnp.reshape(bidx_ref[pl.ds(i, 1), :], (-1,))

        @pl.when(i == 0)
        def _z():
            lsum_ref[0] = 0.0

        lsum_ref[0] += jnp.sum(bdist_ref[pl.ds(i, 1), :])

        @pl.when(i == ni - 1)
        def _w():
            loss_ref[...] = jnp.full((1, 1), lsum_ref[0] * scale, jnp.float32)


def _vq_argmin(z, cb):
    n, d = z.shape
    k = cb.shape[0]
    nj = -(-k // _PW)
    kpad = nj * _PW
    if kpad != k:
        cb = jnp.concatenate(
            [cb, jnp.full((kpad - k, d), 1e4, jnp.float32)], axis=0)
    ni = n // _TN
    scale = 1.25 / (n * d)
    zsq, esq = _row_norms(z, cb)
    idx, loss11 = pl.pallas_call(
        functools.partial(_vq_body, scale),
        grid=(nj, ni),
        in_specs=[
            pl.BlockSpec((_TN, d), lambda j, i: (i, 0)),
            pl.BlockSpec((_PW, d), lambda j, i: (j, 0)),
            pl.BlockSpec((_TN,), lambda j, i: (i,)),
            pl.BlockSpec((_PW, 1), lambda j, i: (j, 0)),
        ],
        out_specs=[
            pl.BlockSpec((_TN,), lambda j, i: (i,)),
            pl.BlockSpec((1, 1), lambda j, i: (0, 0)),
        ],
        out_shape=[
            jax.ShapeDtypeStruct((n,), jnp.int32),
            jax.ShapeDtypeStruct((1, 1), jnp.float32),
        ],
        scratch_shapes=[
            pltpu.VMEM((ni, _TN), jnp.float32),
            pltpu.VMEM((ni, _TN), jnp.int32),
            pltpu.VMEM((ni, _TN), jnp.float32),
            pltpu.SMEM((1,), jnp.float32),
        ],
    )(z, cb, zsq, esq)
    return idx, loss11


def _sc_gather(cb, idx):
    info = plsc.get_sparse_core_info()
    nc, ns = info.num_cores, info.num_subcores
    nw = nc * ns
    n = idx.shape[0]
    d = cb.shape[1]
    b_per_w = n // nw
    ch = 256
    mesh = plsc.VectorSubcoreMesh(core_axis_name="c", subcore_axis_name="s")

    @functools.partial(
        pl.kernel, mesh=mesh,
        out_type=jax.ShapeDtypeStruct((n, d), jnp.float32),
        scratch_types=[
            pltpu.VMEM((ch,), jnp.int32),
            pltpu.VMEM((ch, d), jnp.float32),
            pltpu.SemaphoreType.DMA,
        ],
    )
    def gk(cb_hbm, idx_hbm, out_hbm, idx_v, rows_v, sem):
        wid = lax.axis_index("s") * nc + lax.axis_index("c")
        for c in range(b_per_w // ch):
            base = wid * b_per_w + c * ch
            pltpu.sync_copy(idx_hbm.at[pl.ds(base, ch)], idx_v)
            pltpu.async_copy(cb_hbm.at[idx_v], rows_v, sem).wait()
            pltpu.sync_copy(rows_v, out_hbm.at[pl.ds(base, ch)])

    return gk(cb, idx)


def kernel(motion, codebook):
    b, t, d = motion.shape
    z = motion.reshape(-1, d)
    idx, loss11 = _vq_argmin(z, codebook)
    q = _sc_gather(codebook, idx)
    return q.reshape(b, t, d), idx.reshape(b, t), loss11[0, 0]


# speedup vs baseline: 1.1200x; 1.1200x over previous
import functools

import jax
import jax.numpy as jnp
from jax import lax
from jax.experimental import pallas as pl
from jax.experimental.pallas import tpu as pltpu
from jax.experimental.pallas import tpu_sc as plsc

_TN = 256
_PW = 2736
_CH = 304


def _rn_bf16(x):
    return x.astype(jnp.bfloat16).astype(jnp.float32)


def _rowsum_sq_t(a):
    sq = a * a
    t = sq[:, :128] + sq[:, 128:]
    outs = []
    for g in range(a.shape[0] // 128):
        tg = jnp.transpose(t[128 * g:128 * (g + 1), :])
        acc = tg[0:8]
        for v in range(1, 16):
            acc = acc + tg[8 * v:8 * (v + 1)]
        b = acc[0:4] + acc[4:8]
        c = b[0:2] + b[2:4]
        outs.append(c[0:1] + c[1:2])
    return jnp.concatenate(outs, axis=1)


def _rowsum_sq_narrow(a):
    sq = a * a
    t = sq[:, :128] + sq[:, 128:]
    acc = t[:, 0:8]
    for v in range(1, 16):
        acc = acc + t[:, 8 * v:8 * (v + 1)]
    b = acc[:, 0:4] + acc[:, 4:8]
    c = b[:, 0:2] + b[:, 2:4]
    return c[:, 0:1] + c[:, 1:2]


def _zsq_body(z_ref, o_ref):
    o_ref[...] = jnp.reshape(_rowsum_sq_t(z_ref[...]), (-1,))


def _esq_body(cb_ref, o_ref):
    a = cb_ref[...]
    full = 128 * (a.shape[0] // 128)
    head = jnp.transpose(_rowsum_sq_t(a[:full]))
    tail = _rowsum_sq_narrow(a[full:])
    o_ref[...] = jnp.concatenate([head, tail], axis=0)


def _row_norms(z, cbp):
    n, d = z.shape
    kp = cbp.shape[0]
    zsq = pl.pallas_call(
        _zsq_body,
        grid=(n // _TN,),
        in_specs=[pl.BlockSpec((_TN, d), lambda i: (i, 0))],
        out_specs=pl.BlockSpec((_TN,), lambda i: (i,)),
        out_shape=jax.ShapeDtypeStruct((n,), jnp.float32),
    )(z)
    esq = pl.pallas_call(
        _esq_body,
        grid=(kp // _PW,),
        in_specs=[pl.BlockSpec((_PW, d), lambda i: (i, 0))],
        out_specs=pl.BlockSpec((_PW, 1), lambda i: (i, 0)),
        out_shape=jax.ShapeDtypeStruct((kp, 1), jnp.float32),
    )(cbp)
    return zsq, esq


def _vq_body(scale, z_ref, cb_ref, zsq_ref, esq_ref, idx_ref, loss_ref,
             bval_ref, bidx_ref, bdist_ref, lsum_ref):
    j = pl.program_id(0)
    i = pl.program_id(1)
    nj = pl.num_programs(0)
    ni = pl.num_programs(1)

    z = z_ref[...]
    zsq = zsq_ref[...][None, :]

    m_run = None
    gi_run = None
    io = lax.broadcasted_iota(jnp.int32, (_CH, _TN), 0)
    for c in range(_PW // _CH):
        cbc = cb_ref[pl.ds(c * _CH, _CH), :]
        esq_c = esq_ref[pl.ds(c * _CH, _CH), :]
        mm = lax.dot_general(cbc, z, (((1,), (1,)), ((), ())),
                             preferred_element_type=jnp.float32)
        dist = (zsq + esq_c) - 2.0 * mm
        m_c = jnp.min(dist, axis=0)
        gi_c = jnp.min(jnp.where(dist == m_c[None, :], io, jnp.int32(_CH)),
                       axis=0) + (j * _PW + c * _CH)
        if m_run is None:
            m_run, gi_run = m_c, gi_c
        else:
            upd = m_c < m_run
            gi_run = jnp.where(upd, gi_c, gi_run)
            m_run = jnp.where(upd, m_c, m_run)

    @pl.when(j == 0)
    def _first():
        bval_ref[pl.ds(i, 1), :] = _rn_bf16(m_run)[None, :]
        bidx_ref[pl.ds(i, 1), :] = gi_run[None, :]
        bdist_ref[pl.ds(i, 1), :] = m_run[None, :]

    @pl.when(j != 0)
    def _merge():
        av = bval_ref[pl.ds(i, 1), :]
        bi = bidx_ref[pl.ds(i, 1), :]
        m2 = m_run[None, :]
        gi2 = gi_run[None, :]
        better = m2 < av
        take = better | ((m2 == av) & (gi2 < bi))
        bidx_ref[pl.ds(i, 1), :] = jnp.where(take, gi2, bi)
        bdist_ref[pl.ds(i, 1), :] = jnp.where(take, m2,
                                              bdist_ref[pl.ds(i, 1), :])
        bval_ref[pl.ds(i, 1), :] = jnp.where(better, _rn_bf16(m2), av)

    @pl.when(j == nj - 1)
    def _fin():
        idx_ref[...] = jnp.reshape(bidx_ref[pl.ds(i, 1), :], (-1,))

        @pl.when(i == 0)
        def _z():
            lsum_ref[0] = 0.0

        lsum_ref[0] += jnp.sum(bdist_ref[pl.ds(i, 1), :])

        @pl.when(i == ni - 1)
        def _w():
            loss_ref[...] = jnp.full((1, 1), lsum_ref[0] * scale, jnp.float32)


def _vq_argmin(z, cb):
    n, d = z.shape
    k = cb.shape[0]
    nj = -(-k // _PW)
    kpad = nj * _PW
    if kpad != k:
        cb = jnp.concatenate(
            [cb, jnp.full((kpad - k, d), 1e4, jnp.float32)], axis=0)
    ni = n // _TN
    scale = 1.25 / (n * d)
    zsq, esq = _row_norms(z, cb)
    idx, loss11 = pl.pallas_call(
        functools.partial(_vq_body, scale),
        grid=(nj, ni),
        in_specs=[
            pl.BlockSpec((_TN, d), lambda j, i: (i, 0)),
            pl.BlockSpec((_PW, d), lambda j, i: (j, 0)),
            pl.BlockSpec((_TN,), lambda j, i: (i,)),
            pl.BlockSpec((_PW, 1), lambda j, i: (j, 0)),
        ],
        out_specs=[
            pl.BlockSpec((_TN,), lambda j, i: (i,)),
            pl.BlockSpec((1, 1), lambda j, i: (0, 0)),
        ],
        out_shape=[
            jax.ShapeDtypeStruct((n,), jnp.int32),
            jax.ShapeDtypeStruct((1, 1), jnp.float32),
        ],
        scratch_shapes=[
            pltpu.VMEM((ni, _TN), jnp.float32),
            pltpu.VMEM((ni, _TN), jnp.int32),
            pltpu.VMEM((ni, _TN), jnp.float32),
            pltpu.SMEM((1,), jnp.float32),
        ],
    )(z, cb, zsq, esq)
    return idx, loss11


def _sc_gather(cb, idx):
    info = plsc.get_sparse_core_info()
    nc, ns = info.num_cores, info.num_subcores
    nw = nc * ns
    n = idx.shape[0]
    d = cb.shape[1]
    b_per_w = n // nw
    ch = 256
    mesh = plsc.VectorSubcoreMesh(core_axis_name="c", subcore_axis_name="s")

    @functools.partial(
        pl.kernel, mesh=mesh,
        out_type=jax.ShapeDtypeStruct((n, d), jnp.float32),
        scratch_types=[
            pltpu.VMEM((ch,), jnp.int32),
            pltpu.VMEM((ch, d), jnp.float32),
            pltpu.SemaphoreType.DMA,
        ],
    )
    def gk(cb_hbm, idx_hbm, out_hbm, idx_v, rows_v, sem):
        wid = lax.axis_index("s") * nc + lax.axis_index("c")
        for c in range(b_per_w // ch):
            base = wid * b_per_w + c * ch
            pltpu.sync_copy(idx_hbm.at[pl.ds(base, ch)], idx_v)
            pltpu.async_copy(cb_hbm.at[idx_v], rows_v, sem).wait()
            pltpu.sync_copy(rows_v, out_hbm.at[pl.ds(base, ch)])

    return gk(cb, idx)


def kernel(motion, codebook):
    b, t, d = motion.shape
    z = motion.reshape(-1, d)
    idx, loss11 = _vq_argmin(z, codebook)
    q = _sc_gather(codebook, idx)
    return q.reshape(b, t, d), idx.reshape(b, t), loss11[0, 0]


# speedup vs baseline: 1.2179x; 1.0875x over previous
import functools

import jax
import jax.numpy as jnp
from jax import lax
from jax.experimental import pallas as pl
from jax.experimental.pallas import tpu as pltpu
from jax.experimental.pallas import tpu_sc as plsc

_TN = 256
_PW = 2736
_CH = 304


def _rn_bf16(x):
    return x.astype(jnp.bfloat16).astype(jnp.float32)


def _rowsum_sq_t(a):
    sq = a * a
    t = sq[:, :128] + sq[:, 128:]
    outs = []
    for g in range(a.shape[0] // 128):
        tg = jnp.transpose(t[128 * g:128 * (g + 1), :])
        acc = tg[0:8]
        for v in range(1, 16):
            acc = acc + tg[8 * v:8 * (v + 1)]
        b = acc[0:4] + acc[4:8]
        c = b[0:2] + b[2:4]
        outs.append(c[0:1] + c[1:2])
    return jnp.concatenate(outs, axis=1)


def _rowsum_sq_narrow(a):
    sq = a * a
    t = sq[:, :128] + sq[:, 128:]
    acc = t[:, 0:8]
    for v in range(1, 16):
        acc = acc + t[:, 8 * v:8 * (v + 1)]
    b = acc[:, 0:4] + acc[:, 4:8]
    c = b[:, 0:2] + b[:, 2:4]
    return c[:, 0:1] + c[:, 1:2]


def _norms_body(z_ref, cb_ref, zsq_ref, esq_ref):
    zsq_ref[...] = jnp.reshape(_rowsum_sq_t(z_ref[...]), (-1,))
    a = cb_ref[...]
    full = 128 * (a.shape[0] // 128)
    head = jnp.transpose(_rowsum_sq_t(a[:full]))
    tail = _rowsum_sq_narrow(a[full:])
    esq_ref[...] = jnp.concatenate([head, tail], axis=0)


def _row_norms(z, cbp):
    n, d = z.shape
    kp = cbp.shape[0]
    return pl.pallas_call(
        _norms_body,
        out_shape=[
            jax.ShapeDtypeStruct((n,), jnp.float32),
            jax.ShapeDtypeStruct((kp, 1), jnp.float32),
        ],
    )(z, cbp)


def _vq_body(scale, z_ref, cb_ref, zsq_ref, esq_ref, idx_ref, loss_ref,
             bval_ref, bidx_ref, bdist_ref, lsum_ref):
    j = pl.program_id(0)
    i = pl.program_id(1)
    nj = pl.num_programs(0)
    ni = pl.num_programs(1)

    z = z_ref[...]
    zsq = zsq_ref[...][None, :]

    m_run = None
    gi_run = None
    io = lax.broadcasted_iota(jnp.int32, (_CH, _TN), 0)
    for c in range(_PW // _CH):
        cbc = cb_ref[pl.ds(c * _CH, _CH), :]
        esq_c = esq_ref[pl.ds(c * _CH, _CH), :]
        mm = lax.dot_general(cbc, z, (((1,), (1,)), ((), ())),
                             preferred_element_type=jnp.float32)
        dist = (zsq + esq_c) - 2.0 * mm
        m_c = jnp.min(dist, axis=0)
        gi_c = jnp.min(jnp.where(dist == m_c[None, :], io, jnp.int32(_CH)),
                       axis=0) + (j * _PW + c * _CH)
        if m_run is None:
            m_run, gi_run = m_c, gi_c
        else:
            upd = m_c < m_run
            gi_run = jnp.where(upd, gi_c, gi_run)
            m_run = jnp.where(upd, m_c, m_run)

    @pl.when(j == 0)
    def _first():
        bval_ref[pl.ds(i, 1), :] = _rn_bf16(m_run)[None, :]
        bidx_ref[pl.ds(i, 1), :] = gi_run[None, :]
        bdist_ref[pl.ds(i, 1), :] = m_run[None, :]

    @pl.when(j != 0)
    def _merge():
        av = bval_ref[pl.ds(i, 1), :]
        bi = bidx_ref[pl.ds(i, 1), :]
        m2 = m_run[None, :]
        gi2 = gi_run[None, :]
        better = m2 < av
        take = better | ((m2 == av) & (gi2 < bi))
        bidx_ref[pl.ds(i, 1), :] = jnp.where(take, gi2, bi)
        bdist_ref[pl.ds(i, 1), :] = jnp.where(take, m2,
                                              bdist_ref[pl.ds(i, 1), :])
        bval_ref[pl.ds(i, 1), :] = jnp.where(better, _rn_bf16(m2), av)

    @pl.when(j == nj - 1)
    def _fin():
        idx_ref[...] = jnp.reshape(bidx_ref[pl.ds(i, 1), :], (-1,))

        @pl.when(i == 0)
        def _z():
            lsum_ref[0] = 0.0

        lsum_ref[0] += jnp.sum(bdist_ref[pl.ds(i, 1), :])

        @pl.when(i == ni - 1)
        def _w():
            loss_ref[...] = jnp.full((1, 1), lsum_ref[0] * scale, jnp.float32)


def _vq_argmin(z, cb):
    n, d = z.shape
    k = cb.shape[0]
    nj = -(-k // _PW)
    kpad = nj * _PW
    if kpad != k:
        cb = jnp.concatenate(
            [cb, jnp.full((kpad - k, d), 1e4, jnp.float32)], axis=0)
    ni = n // _TN
    scale = 1.25 / (n * d)
    zsq, esq = _row_norms(z, cb)
    idx, loss11 = pl.pallas_call(
        functools.partial(_vq_body, scale),
        grid=(nj, ni),
        in_specs=[
            pl.BlockSpec((_TN, d), lambda j, i: (i, 0)),
            pl.BlockSpec((_PW, d), lambda j, i: (j, 0)),
            pl.BlockSpec((_TN,), lambda j, i: (i,)),
            pl.BlockSpec((_PW, 1), lambda j, i: (j, 0)),
        ],
        out_specs=[
            pl.BlockSpec((_TN,), lambda j, i: (i,)),
            pl.BlockSpec((1, 1), lambda j, i: (0, 0)),
        ],
        out_shape=[
            jax.ShapeDtypeStruct((n,), jnp.int32),
            jax.ShapeDtypeStruct((1, 1), jnp.float32),
        ],
        scratch_shapes=[
            pltpu.VMEM((ni, _TN), jnp.float32),
            pltpu.VMEM((ni, _TN), jnp.int32),
            pltpu.VMEM((ni, _TN), jnp.float32),
            pltpu.SMEM((1,), jnp.float32),
        ],
    )(z, cb, zsq, esq)
    return idx, loss11


def _sc_gather(cb, idx):
    info = plsc.get_sparse_core_info()
    nc, ns = info.num_cores, info.num_subcores
    nw = nc * ns
    n = idx.shape[0]
    d = cb.shape[1]
    b_per_w = n // nw
    ch = 256
    mesh = plsc.VectorSubcoreMesh(core_axis_name="c", subcore_axis_name="s")

    @functools.partial(
        pl.kernel, mesh=mesh,
        out_type=jax.ShapeDtypeStruct((n, d), jnp.float32),
        scratch_types=[
            pltpu.VMEM((ch,), jnp.int32),
            pltpu.VMEM((ch, d), jnp.float32),
            pltpu.SemaphoreType.DMA,
        ],
    )
    def gk(cb_hbm, idx_hbm, out_hbm, idx_v, rows_v, sem):
        wid = lax.axis_index("s") * nc + lax.axis_index("c")
        for c in range(b_per_w // ch):
            base = wid * b_per_w + c * ch
            pltpu.sync_copy(idx_hbm.at[pl.ds(base, ch)], idx_v)
            pltpu.async_copy(cb_hbm.at[idx_v], rows_v, sem).wait()
            pltpu.sync_copy(rows_v, out_hbm.at[pl.ds(base, ch)])

    return gk(cb, idx)


def kernel(motion, codebook):
    b, t, d = motion.shape
    z = motion.reshape(-1, d)
    idx, loss11 = _vq_argmin(z, codebook)
    q = _sc_gather(codebook, idx)
    return q.reshape(b, t, d), idx.reshape(b, t), loss11[0, 0]


# speedup vs baseline: 1.4042x; 1.1529x over previous
import functools

import jax
import jax.numpy as jnp
from jax import lax
from jax.experimental import pallas as pl
from jax.experimental.pallas import tpu as pltpu
from jax.experimental.pallas import tpu_sc as plsc

_TN = 512
_PW = 2736
_CH = 304


def _rn_bf16(x):
    return x.astype(jnp.bfloat16).astype(jnp.float32)


def _rowsum_sq_t(a):
    sq = a * a
    t = sq[:, :128] + sq[:, 128:]
    outs = []
    for g in range(a.shape[0] // 128):
        tg = jnp.transpose(t[128 * g:128 * (g + 1), :])
        acc = tg[0:8]
        for v in range(1, 16):
            acc = acc + tg[8 * v:8 * (v + 1)]
        b = acc[0:4] + acc[4:8]
        c = b[0:2] + b[2:4]
        outs.append(c[0:1] + c[1:2])
    return jnp.concatenate(outs, axis=1)


def _rowsum_sq_narrow(a):
    sq = a * a
    t = sq[:, :128] + sq[:, 128:]
    acc = t[:, 0:8]
    for v in range(1, 16):
        acc = acc + t[:, 8 * v:8 * (v + 1)]
    b = acc[:, 0:4] + acc[:, 4:8]
    c = b[:, 0:2] + b[:, 2:4]
    return c[:, 0:1] + c[:, 1:2]


def _norms_body(z_ref, cb_ref, zsq_ref, esq_ref):
    zsq_ref[...] = jnp.reshape(_rowsum_sq_t(z_ref[...]), (-1,))
    a = cb_ref[...]
    full = 128 * (a.shape[0] // 128)
    head = jnp.transpose(_rowsum_sq_t(a[:full]))
    tail = _rowsum_sq_narrow(a[full:])
    esq_ref[...] = jnp.concatenate([head, tail], axis=0)


def _row_norms(z, cbp):
    n, d = z.shape
    kp = cbp.shape[0]
    return pl.pallas_call(
        _norms_body,
        out_shape=[
            jax.ShapeDtypeStruct((n,), jnp.float32),
            jax.ShapeDtypeStruct((kp, 1), jnp.float32),
        ],
    )(z, cbp)


def _vq_body(scale, z_ref, cb_ref, zsq_ref, esq_ref, idx_ref, loss_ref,
             bval_ref, bidx_ref, bdist_ref, lsum_ref):
    j = pl.program_id(0)
    i = pl.program_id(1)
    nj = pl.num_programs(0)
    ni = pl.num_programs(1)

    z = z_ref[...]
    zsq = zsq_ref[...][None, :]

    m_run = None
    gi_run = None
    io = lax.broadcasted_iota(jnp.int32, (_CH, _TN), 0)
    for c in range(_PW // _CH):
        cbc = cb_ref[pl.ds(c * _CH, _CH), :]
        esq_c = esq_ref[pl.ds(c * _CH, _CH), :]
        mm = lax.dot_general(cbc, z, (((1,), (1,)), ((), ())),
                             preferred_element_type=jnp.float32)
        dist = (zsq + esq_c) - 2.0 * mm
        m_c = jnp.min(dist, axis=0)
        gi_c = jnp.min(jnp.where(dist == m_c[None, :], io, jnp.int32(_CH)),
                       axis=0) + (j * _PW + c * _CH)
        if m_run is None:
            m_run, gi_run = m_c, gi_c
        else:
            upd = m_c < m_run
            gi_run = jnp.where(upd, gi_c, gi_run)
            m_run = jnp.where(upd, m_c, m_run)

    @pl.when(j == 0)
    def _first():
        bval_ref[pl.ds(i, 1), :] = _rn_bf16(m_run)[None, :]
        bidx_ref[pl.ds(i, 1), :] = gi_run[None, :]
        bdist_ref[pl.ds(i, 1), :] = m_run[None, :]

    @pl.when(j != 0)
    def _merge():
        av = bval_ref[pl.ds(i, 1), :]
        bi = bidx_ref[pl.ds(i, 1), :]
        m2 = m_run[None, :]
        gi2 = gi_run[None, :]
        better = m2 < av
        take = better | ((m2 == av) & (gi2 < bi))
        bidx_ref[pl.ds(i, 1), :] = jnp.where(take, gi2, bi)
        bdist_ref[pl.ds(i, 1), :] = jnp.where(take, m2,
                                              bdist_ref[pl.ds(i, 1), :])
        bval_ref[pl.ds(i, 1), :] = jnp.where(better, _rn_bf16(m2), av)

    @pl.when(j == nj - 1)
    def _fin():
        idx_ref[...] = jnp.reshape(bidx_ref[pl.ds(i, 1), :], (-1,))

        @pl.when(i == 0)
        def _z():
            lsum_ref[0] = 0.0

        lsum_ref[0] += jnp.sum(bdist_ref[pl.ds(i, 1), :])

        @pl.when(i == ni - 1)
        def _w():
            loss_ref[...] = jnp.full((1, 1), lsum_ref[0] * scale, jnp.float32)


def _vq_argmin(z, cb):
    n, d = z.shape
    k = cb.shape[0]
    nj = -(-k // _PW)
    kpad = nj * _PW
    if kpad != k:
        cb = jnp.concatenate(
            [cb, jnp.full((kpad - k, d), 1e4, jnp.float32)], axis=0)
    ni = n // _TN
    scale = 1.25 / (n * d)
    zsq, esq = _row_norms(z, cb)
    idx, loss11 = pl.pallas_call(
        functools.partial(_vq_body, scale),
        grid=(nj, ni),
        in_specs=[
            pl.BlockSpec((_TN, d), lambda j, i: (i, 0)),
            pl.BlockSpec((_PW, d), lambda j, i: (j, 0)),
            pl.BlockSpec((_TN,), lambda j, i: (i,)),
            pl.BlockSpec((_PW, 1), lambda j, i: (j, 0)),
        ],
        out_specs=[
            pl.BlockSpec((_TN,), lambda j, i: (i,)),
            pl.BlockSpec((1, 1), lambda j, i: (0, 0)),
        ],
        out_shape=[
            jax.ShapeDtypeStruct((n,), jnp.int32),
            jax.ShapeDtypeStruct((1, 1), jnp.float32),
        ],
        scratch_shapes=[
            pltpu.VMEM((ni, _TN), jnp.float32),
            pltpu.VMEM((ni, _TN), jnp.int32),
            pltpu.VMEM((ni, _TN), jnp.float32),
            pltpu.SMEM((1,), jnp.float32),
        ],
    )(z, cb, zsq, esq)
    return idx, loss11


def _sc_gather(cb, idx):
    info = plsc.get_sparse_core_info()
    nc, ns = info.num_cores, info.num_subcores
    nw = nc * ns
    n = idx.shape[0]
    d = cb.shape[1]
    b_per_w = n // nw
    ch = 256
    mesh = plsc.VectorSubcoreMesh(core_axis_name="c", subcore_axis_name="s")

    @functools.partial(
        pl.kernel, mesh=mesh,
        out_type=jax.ShapeDtypeStruct((n, d), jnp.float32),
        scratch_types=[
            pltpu.VMEM((ch,), jnp.int32),
            pltpu.VMEM((ch, d), jnp.float32),
            pltpu.SemaphoreType.DMA,
        ],
    )
    def gk(cb_hbm, idx_hbm, out_hbm, idx_v, rows_v, sem):
        wid = lax.axis_index("s") * nc + lax.axis_index("c")
        for c in range(b_per_w // ch):
            base = wid * b_per_w + c * ch
            pltpu.sync_copy(idx_hbm.at[pl.ds(base, ch)], idx_v)
            pltpu.async_copy(cb_hbm.at[idx_v], rows_v, sem).wait()
            pltpu.sync_copy(rows_v, out_hbm.at[pl.ds(base, ch)])

    return gk(cb, idx)


def kernel(motion, codebook):
    b, t, d = motion.shape
    z = motion.reshape(-1, d)
    idx, loss11 = _vq_argmin(z, codebook)
    q = _sc_gather(codebook, idx)
    return q.reshape(b, t, d), idx.reshape(b, t), loss11[0, 0]


# speedup vs baseline: 1.4422x; 1.0271x over previous
import functools

import jax
import jax.numpy as jnp
from jax import lax
from jax.experimental import pallas as pl
from jax.experimental.pallas import tpu as pltpu
from jax.experimental.pallas import tpu_sc as plsc

_TN = 1024
_PW = 2736
_CH = 304


def _rn_bf16(x):
    return x.astype(jnp.bfloat16).astype(jnp.float32)


def _rowsum_sq_t(a):
    sq = a * a
    t = sq[:, :128] + sq[:, 128:]
    outs = []
    for g in range(a.shape[0] // 128):
        tg = jnp.transpose(t[128 * g:128 * (g + 1), :])
        acc = tg[0:8]
        for v in range(1, 16):
            acc = acc + tg[8 * v:8 * (v + 1)]
        b = acc[0:4] + acc[4:8]
        c = b[0:2] + b[2:4]
        outs.append(c[0:1] + c[1:2])
    return jnp.concatenate(outs, axis=1)


def _rowsum_sq_narrow(a):
    sq = a * a
    t = sq[:, :128] + sq[:, 128:]
    acc = t[:, 0:8]
    for v in range(1, 16):
        acc = acc + t[:, 8 * v:8 * (v + 1)]
    b = acc[:, 0:4] + acc[:, 4:8]
    c = b[:, 0:2] + b[:, 2:4]
    return c[:, 0:1] + c[:, 1:2]


def _norms_body(z_ref, cb_ref, zsq_ref, esq_ref):
    zsq_ref[...] = jnp.reshape(_rowsum_sq_t(z_ref[...]), (-1,))
    a = cb_ref[...]
    full = 128 * (a.shape[0] // 128)
    head = jnp.transpose(_rowsum_sq_t(a[:full]))
    tail = _rowsum_sq_narrow(a[full:])
    esq_ref[...] = jnp.concatenate([head, tail], axis=0)


def _row_norms(z, cbp):
    n, d = z.shape
    kp = cbp.shape[0]
    return pl.pallas_call(
        _norms_body,
        out_shape=[
            jax.ShapeDtypeStruct((n,), jnp.float32),
            jax.ShapeDtypeStruct((kp, 1), jnp.float32),
        ],
    )(z, cbp)


def _vq_body(scale, z_ref, cb_ref, zsq_ref, esq_ref, idx_ref, loss_ref,
             bval_ref, bidx_ref, bdist_ref, lsum_ref):
    j = pl.program_id(0)
    i = pl.program_id(1)
    nj = pl.num_programs(0)
    ni = pl.num_programs(1)

    z = z_ref[...]
    zsq = zsq_ref[...][None, :]

    m_run = None
    gi_run = None
    io = lax.broadcasted_iota(jnp.int32, (_CH, _TN), 0)
    for c in range(_PW // _CH):
        cbc = cb_ref[pl.ds(c * _CH, _CH), :]
        esq_c = esq_ref[pl.ds(c * _CH, _CH), :]
        mm = lax.dot_general(cbc, z, (((1,), (1,)), ((), ())),
                             preferred_element_type=jnp.float32)
        dist = (zsq + esq_c) - 2.0 * mm
        m_c = jnp.min(dist, axis=0)
        gi_c = jnp.min(jnp.where(dist == m_c[None, :], io, jnp.int32(_CH)),
                       axis=0) + (j * _PW + c * _CH)
        if m_run is None:
            m_run, gi_run = m_c, gi_c
        else:
            upd = m_c < m_run
            gi_run = jnp.where(upd, gi_c, gi_run)
            m_run = jnp.where(upd, m_c, m_run)

    @pl.when(j == 0)
    def _first():
        bval_ref[pl.ds(i, 1), :] = _rn_bf16(m_run)[None, :]
        bidx_ref[pl.ds(i, 1), :] = gi_run[None, :]
        bdist_ref[pl.ds(i, 1), :] = m_run[None, :]

    @pl.when(j != 0)
    def _merge():
        av = bval_ref[pl.ds(i, 1), :]
        bi = bidx_ref[pl.ds(i, 1), :]
        m2 = m_run[None, :]
        gi2 = gi_run[None, :]
        better = m2 < av
        take = better | ((m2 == av) & (gi2 < bi))
        bidx_ref[pl.ds(i, 1), :] = jnp.where(take, gi2, bi)
        bdist_ref[pl.ds(i, 1), :] = jnp.where(take, m2,
                                              bdist_ref[pl.ds(i, 1), :])
        bval_ref[pl.ds(i, 1), :] = jnp.where(better, _rn_bf16(m2), av)

    @pl.when(j == nj - 1)
    def _fin():
        idx_ref[...] = jnp.reshape(bidx_ref[pl.ds(i, 1), :], (-1,))

        @pl.when(i == 0)
        def _z():
            lsum_ref[0] = 0.0

        lsum_ref[0] += jnp.sum(bdist_ref[pl.ds(i, 1), :])

        @pl.when(i == ni - 1)
        def _w():
            loss_ref[...] = jnp.full((1, 1), lsum_ref[0] * scale, jnp.float32)


def _vq_argmin(z, cb):
    n, d = z.shape
    k = cb.shape[0]
    nj = -(-k // _PW)
    kpad = nj * _PW
    if kpad != k:
        cb = jnp.concatenate(
            [cb, jnp.full((kpad - k, d), 1e4, jnp.float32)], axis=0)
    ni = n // _TN
    scale = 1.25 / (n * d)
    zsq, esq = _row_norms(z, cb)
    idx, loss11 = pl.pallas_call(
        functools.partial(_vq_body, scale),
        grid=(nj, ni),
        in_specs=[
            pl.BlockSpec((_TN, d), lambda j, i: (i, 0)),
            pl.BlockSpec((_PW, d), lambda j, i: (j, 0)),
            pl.BlockSpec((_TN,), lambda j, i: (i,)),
            pl.BlockSpec((_PW, 1), lambda j, i: (j, 0)),
        ],
        out_specs=[
            pl.BlockSpec((_TN,), lambda j, i: (i,)),
            pl.BlockSpec((1, 1), lambda j, i: (0, 0)),
        ],
        out_shape=[
            jax.ShapeDtypeStruct((n,), jnp.int32),
            jax.ShapeDtypeStruct((1, 1), jnp.float32),
        ],
        scratch_shapes=[
            pltpu.VMEM((ni, _TN), jnp.float32),
            pltpu.VMEM((ni, _TN), jnp.int32),
            pltpu.VMEM((ni, _TN), jnp.float32),
            pltpu.SMEM((1,), jnp.float32),
        ],
    )(z, cb, zsq, esq)
    return idx, loss11


def _sc_gather(cb, idx):
    info = plsc.get_sparse_core_info()
    nc, ns = info.num_cores, info.num_subcores
    nw = nc * ns
    n = idx.shape[0]
    d = cb.shape[1]
    b_per_w = n // nw
    ch = 256
    mesh = plsc.VectorSubcoreMesh(core_axis_name="c", subcore_axis_name="s")

    @functools.partial(
        pl.kernel, mesh=mesh,
        out_type=jax.ShapeDtypeStruct((n, d), jnp.float32),
        scratch_types=[
            pltpu.VMEM((ch,), jnp.int32),
            pltpu.VMEM((ch, d), jnp.float32),
            pltpu.SemaphoreType.DMA,
        ],
    )
    def gk(cb_hbm, idx_hbm, out_hbm, idx_v, rows_v, sem):
        wid = lax.axis_index("s") * nc + lax.axis_index("c")
        for c in range(b_per_w // ch):
            base = wid * b_per_w + c * ch
            pltpu.sync_copy(idx_hbm.at[pl.ds(base, ch)], idx_v)
            pltpu.async_copy(cb_hbm.at[idx_v], rows_v, sem).wait()
            pltpu.sync_copy(rows_v, out_hbm.at[pl.ds(base, ch)])

    return gk(cb, idx)


def kernel(motion, codebook):
    b, t, d = motion.shape
    z = motion.reshape(-1, d)
    idx, loss11 = _vq_argmin(z, codebook)
    q = _sc_gather(codebook, idx)
    return q.reshape(b, t, d), idx.reshape(b, t), loss11[0, 0]


# speedup vs baseline: 1.5271x; 1.0589x over previous
import functools

import jax
import jax.numpy as jnp
from jax import lax
from jax.experimental import pallas as pl
from jax.experimental.pallas import tpu as pltpu
from jax.experimental.pallas import tpu_sc as plsc

_TN = 2048
_PW = 2736
_CH = 304


def _rn_bf16(x):
    return x.astype(jnp.bfloat16).astype(jnp.float32)


def _rowsum_sq_t(a):
    sq = a * a
    t = sq[:, :128] + sq[:, 128:]
    outs = []
    for g in range(a.shape[0] // 128):
        tg = jnp.transpose(t[128 * g:128 * (g + 1), :])
        acc = tg[0:8]
        for v in range(1, 16):
            acc = acc + tg[8 * v:8 * (v + 1)]
        b = acc[0:4] + acc[4:8]
        c = b[0:2] + b[2:4]
        outs.append(c[0:1] + c[1:2])
    return jnp.concatenate(outs, axis=1)


def _rowsum_sq_narrow(a):
    sq = a * a
    t = sq[:, :128] + sq[:, 128:]
    acc = t[:, 0:8]
    for v in range(1, 16):
        acc = acc + t[:, 8 * v:8 * (v + 1)]
    b = acc[:, 0:4] + acc[:, 4:8]
    c = b[:, 0:2] + b[:, 2:4]
    return c[:, 0:1] + c[:, 1:2]


def _norms_body(z_ref, cb_ref, zsq_ref, esq_ref):
    zsq_ref[...] = jnp.reshape(_rowsum_sq_t(z_ref[...]), (-1,))
    a = cb_ref[...]
    full = 128 * (a.shape[0] // 128)
    head = jnp.transpose(_rowsum_sq_t(a[:full]))
    tail = _rowsum_sq_narrow(a[full:])
    esq_ref[...] = jnp.concatenate([head, tail], axis=0)


def _row_norms(z, cbp):
    n, d = z.shape
    kp = cbp.shape[0]
    return pl.pallas_call(
        _norms_body,
        out_shape=[
            jax.ShapeDtypeStruct((n,), jnp.float32),
            jax.ShapeDtypeStruct((kp, 1), jnp.float32),
        ],
    )(z, cbp)


def _vq_body(scale, z_ref, cb_ref, zsq_ref, esq_ref, idx_ref, loss_ref,
             bval_ref, bidx_ref, bdist_ref, lsum_ref):
    j = pl.program_id(0)
    i = pl.program_id(1)
    nj = pl.num_programs(0)
    ni = pl.num_programs(1)

    z = z_ref[...]
    zsq = zsq_ref[...][None, :]

    m_run = None
    gi_run = None
    io = lax.broadcasted_iota(jnp.int32, (_CH, _TN), 0)
    for c in range(_PW // _CH):
        cbc = cb_ref[pl.ds(c * _CH, _CH), :]
        esq_c = esq_ref[pl.ds(c * _CH, _CH), :]
        mm = lax.dot_general(cbc, z, (((1,), (1,)), ((), ())),
                             preferred_element_type=jnp.float32)
        dist = (zsq + esq_c) - 2.0 * mm
        m_c = jnp.min(dist, axis=0)
        gi_c = jnp.min(jnp.where(dist == m_c[None, :], io, jnp.int32(_CH)),
                       axis=0) + (j * _PW + c * _CH)
        if m_run is None:
            m_run, gi_run = m_c, gi_c
        else:
            upd = m_c < m_run
            gi_run = jnp.where(upd, gi_c, gi_run)
            m_run = jnp.where(upd, m_c, m_run)

    @pl.when(j == 0)
    def _first():
        bval_ref[pl.ds(i, 1), :] = _rn_bf16(m_run)[None, :]
        bidx_ref[pl.ds(i, 1), :] = gi_run[None, :]
        bdist_ref[pl.ds(i, 1), :] = m_run[None, :]

    @pl.when(j != 0)
    def _merge():
        av = bval_ref[pl.ds(i, 1), :]
        bi = bidx_ref[pl.ds(i, 1), :]
        m2 = m_run[None, :]
        gi2 = gi_run[None, :]
        better = m2 < av
        take = better | ((m2 == av) & (gi2 < bi))
        bidx_ref[pl.ds(i, 1), :] = jnp.where(take, gi2, bi)
        bdist_ref[pl.ds(i, 1), :] = jnp.where(take, m2,
                                              bdist_ref[pl.ds(i, 1), :])
        bval_ref[pl.ds(i, 1), :] = jnp.where(better, _rn_bf16(m2), av)

    @pl.when(j == nj - 1)
    def _fin():
        idx_ref[...] = jnp.reshape(bidx_ref[pl.ds(i, 1), :], (-1,))

        @pl.when(i == 0)
        def _z():
            lsum_ref[0] = 0.0

        lsum_ref[0] += jnp.sum(bdist_ref[pl.ds(i, 1), :])

        @pl.when(i == ni - 1)
        def _w():
            loss_ref[...] = jnp.full((1, 1), lsum_ref[0] * scale, jnp.float32)


def _vq_argmin(z, cb):
    n, d = z.shape
    k = cb.shape[0]
    nj = -(-k // _PW)
    kpad = nj * _PW
    if kpad != k:
        cb = jnp.concatenate(
            [cb, jnp.full((kpad - k, d), 1e4, jnp.float32)], axis=0)
    ni = n // _TN
    scale = 1.25 / (n * d)
    zsq, esq = _row_norms(z, cb)
    idx, loss11 = pl.pallas_call(
        functools.partial(_vq_body, scale),
        grid=(nj, ni),
        in_specs=[
            pl.BlockSpec((_TN, d), lambda j, i: (i, 0)),
            pl.BlockSpec((_PW, d), lambda j, i: (j, 0)),
            pl.BlockSpec((_TN,), lambda j, i: (i,)),
            pl.BlockSpec((_PW, 1), lambda j, i: (j, 0)),
        ],
        out_specs=[
            pl.BlockSpec((_TN,), lambda j, i: (i,)),
            pl.BlockSpec((1, 1), lambda j, i: (0, 0)),
        ],
        out_shape=[
            jax.ShapeDtypeStruct((n,), jnp.int32),
            jax.ShapeDtypeStruct((1, 1), jnp.float32),
        ],
        scratch_shapes=[
            pltpu.VMEM((ni, _TN), jnp.float32),
            pltpu.VMEM((ni, _TN), jnp.int32),
            pltpu.VMEM((ni, _TN), jnp.float32),
            pltpu.SMEM((1,), jnp.float32),
        ],
    )(z, cb, zsq, esq)
    return idx, loss11


def _sc_gather(cb, idx):
    info = plsc.get_sparse_core_info()
    nc, ns = info.num_cores, info.num_subcores
    nw = nc * ns
    n = idx.shape[0]
    d = cb.shape[1]
    b_per_w = n // nw
    ch = 256
    mesh = plsc.VectorSubcoreMesh(core_axis_name="c", subcore_axis_name="s")

    @functools.partial(
        pl.kernel, mesh=mesh,
        out_type=jax.ShapeDtypeStruct((n, d), jnp.float32),
        scratch_types=[
            pltpu.VMEM((ch,), jnp.int32),
            pltpu.VMEM((ch, d), jnp.float32),
            pltpu.SemaphoreType.DMA,
        ],
    )
    def gk(cb_hbm, idx_hbm, out_hbm, idx_v, rows_v, sem):
        wid = lax.axis_index("s") * nc + lax.axis_index("c")
        for c in range(b_per_w // ch):
            base = wid * b_per_w + c * ch
            pltpu.sync_copy(idx_hbm.at[pl.ds(base, ch)], idx_v)
            pltpu.async_copy(cb_hbm.at[idx_v], rows_v, sem).wait()
            pltpu.sync_copy(rows_v, out_hbm.at[pl.ds(base, ch)])

    return gk(cb, idx)


def kernel(motion, codebook):
    b, t, d = motion.shape
    z = motion.reshape(-1, d)
    idx, loss11 = _vq_argmin(z, codebook)
    q = _sc_gather(codebook, idx)
    return q.reshape(b, t, d), idx.reshape(b, t), loss11[0, 0]


# speedup vs baseline: 1.5592x; 1.0210x over previous
import functools

import jax
import jax.numpy as jnp
from jax import lax
from jax.experimental import pallas as pl
from jax.experimental.pallas import tpu as pltpu
from jax.experimental.pallas import tpu_sc as plsc

_TN = 4096
_PW = 2736
_CH = 304


def _rn_bf16(x):
    return x.astype(jnp.bfloat16).astype(jnp.float32)


def _rowsum_sq_t(a):
    sq = a * a
    t = sq[:, :128] + sq[:, 128:]
    outs = []
    for g in range(a.shape[0] // 128):
        tg = jnp.transpose(t[128 * g:128 * (g + 1), :])
        acc = tg[0:8]
        for v in range(1, 16):
            acc = acc + tg[8 * v:8 * (v + 1)]
        b = acc[0:4] + acc[4:8]
        c = b[0:2] + b[2:4]
        outs.append(c[0:1] + c[1:2])
    return jnp.concatenate(outs, axis=1)


def _rowsum_sq_narrow(a):
    sq = a * a
    t = sq[:, :128] + sq[:, 128:]
    acc = t[:, 0:8]
    for v in range(1, 16):
        acc = acc + t[:, 8 * v:8 * (v + 1)]
    b = acc[:, 0:4] + acc[:, 4:8]
    c = b[:, 0:2] + b[:, 2:4]
    return c[:, 0:1] + c[:, 1:2]


def _norms_body(z_ref, cb_ref, zsq_ref, esq_ref):
    zsq_ref[...] = jnp.reshape(_rowsum_sq_t(z_ref[...]), (-1,))
    a = cb_ref[...]
    full = 128 * (a.shape[0] // 128)
    head = jnp.transpose(_rowsum_sq_t(a[:full]))
    tail = _rowsum_sq_narrow(a[full:])
    esq_ref[...] = jnp.concatenate([head, tail], axis=0)


def _row_norms(z, cbp):
    n, d = z.shape
    kp = cbp.shape[0]
    return pl.pallas_call(
        _norms_body,
        out_shape=[
            jax.ShapeDtypeStruct((n,), jnp.float32),
            jax.ShapeDtypeStruct((kp, 1), jnp.float32),
        ],
    )(z, cbp)


def _vq_body(scale, z_ref, cb_ref, zsq_ref, esq_ref, idx_ref, loss_ref,
             bval_ref, bidx_ref, bdist_ref, lsum_ref):
    j = pl.program_id(0)
    i = pl.program_id(1)
    nj = pl.num_programs(0)
    ni = pl.num_programs(1)

    z = z_ref[...]
    zsq = zsq_ref[...][None, :]

    m_run = None
    gi_run = None
    io = lax.broadcasted_iota(jnp.int32, (_CH, _TN), 0)
    for c in range(_PW // _CH):
        cbc = cb_ref[pl.ds(c * _CH, _CH), :]
        esq_c = esq_ref[pl.ds(c * _CH, _CH), :]
        mm = lax.dot_general(cbc, z, (((1,), (1,)), ((), ())),
                             preferred_element_type=jnp.float32)
        dist = (zsq + esq_c) - 2.0 * mm
        m_c = jnp.min(dist, axis=0)
        gi_c = jnp.min(jnp.where(dist == m_c[None, :], io, jnp.int32(_CH)),
                       axis=0) + (j * _PW + c * _CH)
        if m_run is None:
            m_run, gi_run = m_c, gi_c
        else:
            upd = m_c < m_run
            gi_run = jnp.where(upd, gi_c, gi_run)
            m_run = jnp.where(upd, m_c, m_run)

    @pl.when(j == 0)
    def _first():
        bval_ref[pl.ds(i, 1), :] = _rn_bf16(m_run)[None, :]
        bidx_ref[pl.ds(i, 1), :] = gi_run[None, :]
        bdist_ref[pl.ds(i, 1), :] = m_run[None, :]

    @pl.when(j != 0)
    def _merge():
        av = bval_ref[pl.ds(i, 1), :]
        bi = bidx_ref[pl.ds(i, 1), :]
        m2 = m_run[None, :]
        gi2 = gi_run[None, :]
        better = m2 < av
        take = better | ((m2 == av) & (gi2 < bi))
        bidx_ref[pl.ds(i, 1), :] = jnp.where(take, gi2, bi)
        bdist_ref[pl.ds(i, 1), :] = jnp.where(take, m2,
                                              bdist_ref[pl.ds(i, 1), :])
        bval_ref[pl.ds(i, 1), :] = jnp.where(better, _rn_bf16(m2), av)

    @pl.when(j == nj - 1)
    def _fin():
        idx_ref[...] = jnp.reshape(bidx_ref[pl.ds(i, 1), :], (-1,))

        @pl.when(i == 0)
        def _z():
            lsum_ref[0] = 0.0

        lsum_ref[0] += jnp.sum(bdist_ref[pl.ds(i, 1), :])

        @pl.when(i == ni - 1)
        def _w():
            loss_ref[...] = jnp.full((1, 1), lsum_ref[0] * scale, jnp.float32)


def _vq_argmin(z, cb):
    n, d = z.shape
    k = cb.shape[0]
    nj = -(-k // _PW)
    kpad = nj * _PW
    if kpad != k:
        cb = jnp.concatenate(
            [cb, jnp.full((kpad - k, d), 1e4, jnp.float32)], axis=0)
    ni = n // _TN
    scale = 1.25 / (n * d)
    zsq, esq = _row_norms(z, cb)
    idx, loss11 = pl.pallas_call(
        functools.partial(_vq_body, scale),
        grid=(nj, ni),
        in_specs=[
            pl.BlockSpec((_TN, d), lambda j, i: (i, 0)),
            pl.BlockSpec((_PW, d), lambda j, i: (j, 0)),
            pl.BlockSpec((_TN,), lambda j, i: (i,)),
            pl.BlockSpec((_PW, 1), lambda j, i: (j, 0)),
        ],
        out_specs=[
            pl.BlockSpec((_TN,), lambda j, i: (i,)),
            pl.BlockSpec((1, 1), lambda j, i: (0, 0)),
        ],
        out_shape=[
            jax.ShapeDtypeStruct((n,), jnp.int32),
            jax.ShapeDtypeStruct((1, 1), jnp.float32),
        ],
        scratch_shapes=[
            pltpu.VMEM((ni, _TN), jnp.float32),
            pltpu.VMEM((ni, _TN), jnp.int32),
            pltpu.VMEM((ni, _TN), jnp.float32),
            pltpu.SMEM((1,), jnp.float32),
        ],
    )(z, cb, zsq, esq)
    return idx, loss11


def _sc_gather(cb, idx):
    info = plsc.get_sparse_core_info()
    nc, ns = info.num_cores, info.num_subcores
    nw = nc * ns
    n = idx.shape[0]
    d = cb.shape[1]
    b_per_w = n // nw
    ch = 256
    mesh = plsc.VectorSubcoreMesh(core_axis_name="c", subcore_axis_name="s")

    @functools.partial(
        pl.kernel, mesh=mesh,
        out_type=jax.ShapeDtypeStruct((n, d), jnp.float32),
        scratch_types=[
            pltpu.VMEM((ch,), jnp.int32),
            pltpu.VMEM((ch, d), jnp.float32),
            pltpu.SemaphoreType.DMA,
        ],
    )
    def gk(cb_hbm, idx_hbm, out_hbm, idx_v, rows_v, sem):
        wid = lax.axis_index("s") * nc + lax.axis_index("c")
        for c in range(b_per_w // ch):
            base = wid * b_per_w + c * ch
            pltpu.sync_copy(idx_hbm.at[pl.ds(base, ch)], idx_v)
            pltpu.async_copy(cb_hbm.at[idx_v], rows_v, sem).wait()
            pltpu.sync_copy(rows_v, out_hbm.at[pl.ds(base, ch)])

    return gk(cb, idx)


def kernel(motion, codebook):
    b, t, d = motion.shape
    z = motion.reshape(-1, d)
    idx, loss11 = _vq_argmin(z, codebook)
    q = _sc_gather(codebook, idx)
    return q.reshape(b, t, d), idx.reshape(b, t), loss11[0, 0]
